# Initial kernel scaffold; baseline (speedup 1.0000x reference)
import jax
import jax.numpy as jnp
from jax import lax
from jax.experimental import pallas as pl
from jax.experimental.pallas import tpu as pltpu
from jax.experimental.pallas import tpu_sc as plsc

NC = 2
NS = 16
LANES = 16
NW = NC * NS

B = 16384
C = 50
DIM = 32
CHUNK = 16
B_PER_W = B // NW
N_CHUNKS = B_PER_W // CHUNK


def _mfnet_body(u_idx_hbm, i_idx_hbm, user_hbm, item_hbm, out_hbm,
                u_idx_v, i_idx_v, u_v, item_v, out_v, sem_u, sem_i):
    cid = lax.axis_index("c")
    sid = lax.axis_index("s")
    wid = sid * NC + cid
    base = wid * B_PER_W

    b_iota = lax.iota(jnp.int32, (LANES,))

    def chunk_body(t, carry):
        row0 = base + t * CHUNK
        pltpu.sync_copy(u_idx_hbm.at[pl.ds(row0, CHUNK)], u_idx_v)
        pltpu.sync_copy(i_idx_hbm.at[pl.ds(row0, CHUNK), :], i_idx_v)
        cu = pltpu.async_copy(user_hbm.at[u_idx_v], u_v, sem_u)
        handles = [
            pltpu.async_copy(item_hbm.at[i_idx_v.at[b]], item_v.at[b], sem_i)
            for b in range(CHUNK)
        ]
        cu.wait()
        for h in handles:
            h.wait()

        u_regs = [
            plsc.load_gather(u_v, [b_iota, jnp.full((LANES,), d, jnp.int32)])
            for d in range(DIM)
        ]

        def c_body(c, inner):
            cc = jnp.full((LANES,), c, jnp.int32)
            acc = jnp.zeros((LANES,), jnp.float32)
            for d in range(DIM):
                iv = plsc.load_gather(
                    item_v, [b_iota, cc, jnp.full((LANES,), d, jnp.int32)])
                acc = acc + iv * u_regs[d]
            plsc.store_scatter(out_v, [b_iota, cc], acc)
            return inner

        lax.fori_loop(0, C, c_body, 0)
        pltpu.sync_copy(out_v, out_hbm.at[pl.ds(row0, CHUNK), :])
        return carry

    lax.fori_loop(0, N_CHUNKS, chunk_body, 0)


def kernel(u_idx, i_idx, user_emb, item_emb):
    mesh = plsc.VectorSubcoreMesh(core_axis_name="c", subcore_axis_name="s")
    f = pl.kernel(
        _mfnet_body,
        out_type=jax.ShapeDtypeStruct((B, C), jnp.float32),
        mesh=mesh,
        scratch_types=[
            pltpu.VMEM((CHUNK,), jnp.int32),
            pltpu.VMEM((CHUNK, C), jnp.int32),
            pltpu.VMEM((CHUNK, DIM), jnp.float32),
            pltpu.VMEM((CHUNK, C, DIM), jnp.float32),
            pltpu.VMEM((CHUNK, C), jnp.float32),
            pltpu.SemaphoreType.DMA,
            pltpu.SemaphoreType.DMA,
        ],
    )
    return f(u_idx.astype(jnp.int32), i_idx.astype(jnp.int32),
             user_emb, item_emb)


# speedup vs baseline: 1.2004x; 1.2004x over previous
import jax
import jax.numpy as jnp
from jax import lax
from jax.experimental import pallas as pl
from jax.experimental.pallas import tpu as pltpu
from jax.experimental.pallas import tpu_sc as plsc

NC = 2
NS = 16
LANES = 16
NW = NC * NS

B = 16384
C = 50
DIM = 32
CHUNK = 16
B_PER_W = B // NW
N_CHUNKS = B_PER_W // CHUNK


def _mfnet_body(u_idx_hbm, i_idx_hbm, user_hbm, item_hbm, out_hbm,
                u_idx_v, i_idx_v, u_v, item_v, out_v, sem_u, sem_i):
    cid = lax.axis_index("c")
    sid = lax.axis_index("s")
    wid = sid * NC + cid
    base = wid * B_PER_W

    b_iota = lax.iota(jnp.int32, LANES)

    def chunk_body(t, carry):
        row0 = base + t * CHUNK
        pltpu.sync_copy(u_idx_hbm.at[pl.ds(row0, CHUNK)], u_idx_v)
        pltpu.sync_copy(i_idx_hbm.at[pl.ds(row0, CHUNK), :], i_idx_v)
        cu = pltpu.async_copy(user_hbm.at[u_idx_v], u_v, sem_u)
        handles = [
            pltpu.async_copy(item_hbm.at[i_idx_v.at[b]], item_v.at[b], sem_i)
            for b in range(CHUNK)
        ]
        cu.wait()
        for h in handles:
            h.wait()

        u_regs = [
            plsc.load_gather(u_v, [b_iota, jnp.full((LANES,), d, jnp.int32)])
            for d in range(DIM)
        ]

        def c_body(c, inner):
            cc = jnp.full((LANES,), c, jnp.int32)
            acc = jnp.zeros((LANES,), jnp.float32)
            for d in range(DIM):
                iv = plsc.load_gather(
                    item_v, [b_iota, cc, jnp.full((LANES,), d, jnp.int32)])
                acc = acc + iv * u_regs[d]
            plsc.store_scatter(out_v, [b_iota, cc], acc)
            return inner

        lax.fori_loop(0, C, c_body, 0)
        pltpu.sync_copy(out_v, out_hbm.at[pl.ds(row0, CHUNK), :])
        return carry

    lax.fori_loop(0, N_CHUNKS, chunk_body, 0)


def kernel(u_idx, i_idx, user_emb, item_emb):
    mesh = plsc.VectorSubcoreMesh(core_axis_name="c", subcore_axis_name="s")
    f = pl.kernel(
        _mfnet_body,
        out_type=jax.ShapeDtypeStruct((B, C), jnp.float32),
        mesh=mesh,
        compiler_params=pltpu.CompilerParams(
            use_tc_tiling_on_sc=False,
            needs_layout_passes=False,
        ),
        scratch_types=[
            pltpu.VMEM((CHUNK,), jnp.int32),
            pltpu.VMEM((CHUNK, C), jnp.int32),
            pltpu.VMEM((CHUNK, DIM), jnp.float32),
            pltpu.VMEM((CHUNK, C, DIM), jnp.float32),
            pltpu.VMEM((CHUNK, C), jnp.float32),
            pltpu.SemaphoreType.DMA,
            pltpu.SemaphoreType.DMA,
        ],
    )
    return f(u_idx.astype(jnp.int32), i_idx.astype(jnp.int32),
             user_emb, item_emb)


# speedup vs baseline: 1.3623x; 1.1348x over previous
import jax
import jax.numpy as jnp
from jax import lax
from jax.experimental import pallas as pl
from jax.experimental.pallas import tpu as pltpu
from jax.experimental.pallas import tpu_sc as plsc

NC = 2
NS = 16
LANES = 16
NW = NC * NS

B = 16384
C = 50
DIM = 32
CHUNK = 16
B_PER_W = B // NW
N_CHUNKS = B_PER_W // CHUNK
ROWS = CHUNK * C


def _mfnet_body(u_idx_hbm, i_idx_hbm, user_hbm, item_hbm, out_hbm,
                u_idx_v, i_idx_v, u_v, item_v, out_v,
                sem_u0, sem_u1, sem_i0, sem_i1):
    cid = lax.axis_index("c")
    sid = lax.axis_index("s")
    wid = sid * NC + cid
    base = wid * B_PER_W
    b_iota = lax.iota(jnp.int32, LANES)
    sems_u = (sem_u0, sem_u1)
    sems_i = (sem_i0, sem_i1)

    pltpu.sync_copy(u_idx_hbm.at[pl.ds(base, B_PER_W)], u_idx_v)
    pltpu.sync_copy(i_idx_hbm.at[pl.ds(base, B_PER_W), :], i_idx_v)

    def start(t, buf):
        r0 = t * CHUNK
        pltpu.async_copy(
            user_hbm.at[u_idx_v.at[pl.ds(r0, CHUNK)]],
            u_v.at[buf], sems_u[buf])
        for b in range(CHUNK):
            pltpu.async_copy(
                item_hbm.at[i_idx_v.at[r0 + b]],
                item_v.at[buf, pl.ds(b * C, C)], sems_i[buf])

    def drain(buf):
        pltpu.make_async_copy(
            user_hbm.at[pl.ds(0, CHUNK)], u_v.at[buf], sems_u[buf]).wait()
        pltpu.make_async_copy(
            item_hbm.at[pl.ds(0, ROWS)], item_v.at[buf], sems_i[buf]).wait()

    def compute(t, buf):
        bufv = jnp.full((LANES,), buf, jnp.int32)
        u_regs = [
            plsc.load_gather(
                u_v, [bufv, b_iota, jnp.full((LANES,), d, jnp.int32)])
            for d in range(DIM)
        ]
        row_b = b_iota * C

        def c_body(c, inner):
            rows = row_b + c
            acc = jnp.zeros((LANES,), jnp.float32)
            for d in range(DIM):
                iv = plsc.load_gather(
                    item_v, [bufv, rows, jnp.full((LANES,), d, jnp.int32)])
                acc = acc + iv * u_regs[d]
            plsc.store_scatter(
                out_v, [b_iota, jnp.full((LANES,), c, jnp.int32)], acc)
            return inner

        lax.fori_loop(0, C, c_body, 0)
        pltpu.sync_copy(out_v, out_hbm.at[pl.ds(base + t * CHUNK, CHUNK), :])

    start(0, 0)

    def pair_body(i, carry):
        t0 = i * 2
        start(t0 + 1, 1)
        drain(0)
        compute(t0, 0)

        @pl.when(t0 + 2 < N_CHUNKS)
        def _():
            start(t0 + 2, 0)

        drain(1)
        compute(t0 + 1, 1)
        return carry

    lax.fori_loop(0, N_CHUNKS // 2, pair_body, 0)


def kernel(u_idx, i_idx, user_emb, item_emb):
    mesh = plsc.VectorSubcoreMesh(core_axis_name="c", subcore_axis_name="s")
    f = pl.kernel(
        _mfnet_body,
        out_type=jax.ShapeDtypeStruct((B, C), jnp.float32),
        mesh=mesh,
        compiler_params=pltpu.CompilerParams(
            use_tc_tiling_on_sc=False,
            needs_layout_passes=False,
        ),
        scratch_types=[
            pltpu.VMEM((B_PER_W,), jnp.int32),
            pltpu.VMEM((B_PER_W, C), jnp.int32),
            pltpu.VMEM((2, CHUNK, DIM), jnp.float32),
            pltpu.VMEM((2, ROWS, DIM), jnp.float32),
            pltpu.VMEM((CHUNK, C), jnp.float32),
            pltpu.SemaphoreType.DMA,
            pltpu.SemaphoreType.DMA,
            pltpu.SemaphoreType.DMA,
            pltpu.SemaphoreType.DMA,
        ],
    )
    return f(u_idx.astype(jnp.int32), i_idx.astype(jnp.int32),
             user_emb, item_emb)
